# Initial kernel scaffold; baseline (speedup 1.0000x reference)
import jax
import jax.numpy as jnp
from jax.experimental import pallas as pl


def kernel(node_feats, edge_feats, edge_index, W_src_gate, b_src_gate, W_dst_gate, b_dst_gate, W_edge_gate, b_edge_gate, W_src_update, b_src_update, W_dst_update, b_dst_update, gamma_nodes, beta_nodes, gamma_edges, beta_edges):
    raise NotImplementedError("write your pallas kernel here")


# speedup vs baseline: 1.0999x; 1.0999x over previous
import functools
import jax
import jax.numpy as jnp
from jax.experimental import pallas as pl
from jax.experimental.pallas import tpu as pltpu


def _node_proj_body(x_ref, w4_ref, b4_ref, esrc_ref, edst_ref, bh_ref, cx_ref):
    x = x_ref[...]
    w = w4_ref[...]
    b = b4_ref[...]
    esrc_ref[...] = jnp.dot(x, w[0], preferred_element_type=jnp.float32) + b[0]
    edst_ref[...] = jnp.dot(x, w[1], preferred_element_type=jnp.float32) + b[1]
    bh_ref[...] = jnp.dot(x, w[2], preferred_element_type=jnp.float32) + b[2]
    cx_ref[...] = jnp.dot(x, w[3], preferred_element_type=jnp.float32) + b[3]


def _node_proj(node_feats, Ws, bs, block_rows=2000):
    n, d = node_feats.shape
    w4 = jnp.stack(Ws)
    b4 = jnp.stack(bs)
    grid = (n // block_rows,)
    out = jax.ShapeDtypeStruct((n, d), jnp.float32)
    return pl.pallas_call(
        _node_proj_body,
        grid=grid,
        in_specs=[
            pl.BlockSpec((block_rows, d), lambda i: (i, 0)),
            pl.BlockSpec((4, d, d), lambda i: (0, 0, 0)),
            pl.BlockSpec((4, d), lambda i: (0, 0)),
        ],
        out_specs=[pl.BlockSpec((block_rows, d), lambda i: (i, 0))] * 4,
        out_shape=[out, out, out, out],
    )(node_feats, w4, b4)


def _edge_proj_body(x_ref, w_ref, b_ref, o_ref):
    o_ref[...] = (
        jnp.dot(x_ref[...], w_ref[...], preferred_element_type=jnp.float32)
        + b_ref[...]
    )


def _edge_proj(edge_feats, W, b, block_rows=4000):
    e, d = edge_feats.shape
    grid = (e // block_rows,)
    return pl.pallas_call(
        _edge_proj_body,
        grid=grid,
        in_specs=[
            pl.BlockSpec((block_rows, d), lambda i: (i, 0)),
            pl.BlockSpec((d, d), lambda i: (0, 0)),
            pl.BlockSpec((1, d), lambda i: (0, 0)),
        ],
        out_specs=pl.BlockSpec((block_rows, d), lambda i: (i, 0)),
        out_shape=jax.ShapeDtypeStruct((e, d), jnp.float32),
    )(edge_feats, W, b.reshape(1, d))


def _ln_silu(v, gamma, beta):
    mu = jnp.mean(v, axis=-1, keepdims=True)
    var = jnp.mean(jnp.square(v - mu), axis=-1, keepdims=True)
    t = (v - mu) * jax.lax.rsqrt(var + 1e-5) * gamma + beta
    return t * jax.nn.sigmoid(t)


def _edge_final_body(m_ref, ef_ref, g_ref, bt_ref, y_ref):
    y_ref[...] = ef_ref[...] + _ln_silu(m_ref[...], g_ref[...], bt_ref[...])


def _edge_final(m, edge_feats, gamma, beta, block_rows=4000):
    e, d = m.shape
    grid = (e // block_rows,)
    return pl.pallas_call(
        _edge_final_body,
        grid=grid,
        in_specs=[
            pl.BlockSpec((block_rows, d), lambda i: (i, 0)),
            pl.BlockSpec((block_rows, d), lambda i: (i, 0)),
            pl.BlockSpec((1, d), lambda i: (0, 0)),
            pl.BlockSpec((1, d), lambda i: (0, 0)),
        ],
        out_specs=pl.BlockSpec((block_rows, d), lambda i: (i, 0)),
        out_shape=jax.ShapeDtypeStruct((e, d), jnp.float32),
    )(m, edge_feats, gamma.reshape(1, d), beta.reshape(1, d))


def _node_final_body(cx_ref, ssh_ref, ss_ref, nf_ref, g_ref, bt_ref, x_ref):
    h = ssh_ref[...] / (ss_ref[...] + 1e-6)
    v = cx_ref[...] + h
    x_ref[...] = nf_ref[...] + _ln_silu(v, g_ref[...], bt_ref[...])


def _node_final(cx, ssh, ss, node_feats, gamma, beta, block_rows=2000):
    n, d = cx.shape
    grid = (n // block_rows,)
    blk = pl.BlockSpec((block_rows, d), lambda i: (i, 0))
    vec = pl.BlockSpec((1, d), lambda i: (0, 0))
    return pl.pallas_call(
        _node_final_body,
        grid=grid,
        in_specs=[blk, blk, blk, blk, vec, vec],
        out_specs=blk,
        out_shape=jax.ShapeDtypeStruct((n, d), jnp.float32),
    )(cx, ssh, ss, node_feats, gamma.reshape(1, d), beta.reshape(1, d))


def kernel(node_feats, edge_feats, edge_index,
           W_src_gate, b_src_gate, W_dst_gate, b_dst_gate,
           W_edge_gate, b_edge_gate, W_src_update, b_src_update,
           W_dst_update, b_dst_update,
           gamma_nodes, beta_nodes, gamma_edges, beta_edges):
    n, d = node_feats.shape
    src = edge_index[0]
    dst = edge_index[1]

    e_src, e_dst, bh, cx = _node_proj(
        node_feats,
        [W_src_gate, W_dst_gate, W_dst_update, W_src_update],
        [b_src_gate, b_dst_gate, b_dst_update, b_src_update],
    )
    g = _edge_proj(edge_feats, W_edge_gate, b_edge_gate)

    m = e_src[src] + e_dst[dst] + g
    sigma = jax.nn.sigmoid(m)
    ssh = jax.ops.segment_sum(bh[src] * sigma, dst, num_segments=n)
    ss = jax.ops.segment_sum(sigma, dst, num_segments=n)

    y = _edge_final(m, edge_feats, gamma_edges, beta_edges)
    x = _node_final(cx, ssh, ss, node_feats, gamma_nodes, beta_nodes)
    return (x, y)
